# baseline (speedup 1.0000x reference)
import functools

import jax
import jax.numpy as jnp
import numpy as np
from jax import lax
from jax.experimental import pallas as pl
from jax.experimental.pallas import tpu as pltpu
from jax.experimental.pallas import tpu_sc as plsc

_B, _C, _N, _K, _T = 4, 64, 4096, 16, 128
_HP = lax.Precision.DEFAULT
_BN_SCALE = float(1.0 / np.sqrt(1.0 + 1e-5))

_ROW_W = 128
_OFF_P, _OFF_BV, _OFF_V = 0, 4, 36

_M1 = 256
_M2 = 1024
_M3 = 128


def _knn_body(colsT_ref, rowsT_ref, ptile_ref, out_ref):
    b = pl.program_id(0)
    colsT = colsT_ref[...]
    rows = rowsT_ref[...]
    ptile = ptile_ref[...]
    innerT = lax.dot_general(colsT, rows, (((1,), (1,)), ((), ())),
                             precision=lax.Precision.DEFAULT)
    sq_c = jnp.sum(colsT * colsT, axis=1, keepdims=True)
    sq_q = jnp.sum(ptile * ptile, axis=0, keepdims=True)
    negdT = -((sq_c + sq_q) - 2.0 * innerT)

    S, W, NB = 5, 128, _N // 128
    sub_iota = lax.broadcasted_iota(jnp.int32, (W, _M1), 0)
    vparts, jparts = [], []
    for nb in range(NB):
        seg = negdT[nb * W:(nb + 1) * W, :]
        sv, sj = [], []
        for _ in range(S):
            m = jnp.max(seg, axis=0, keepdims=True)
            cand = jnp.where(seg >= m, sub_iota, W)
            jl = jnp.min(cand, axis=0, keepdims=True)
            sv.append(m)
            sj.append(jl + nb * W)
            seg = jnp.where(cand == jl, -jnp.inf, seg)
        vparts.append(jnp.concatenate(sv, axis=0))
        jparts.append(jnp.concatenate(sj, axis=0))
    V = jnp.concatenate(vparts, axis=0)
    J = jnp.concatenate(jparts, axis=0)
    sel = jnp.zeros(V.shape, jnp.bool_)
    picks = []
    Vw = V
    for _ in range(_K):
        m = jnp.max(Vw, axis=0, keepdims=True)
        cand = jnp.where(Vw >= m, J, _N)
        j = jnp.min(cand, axis=0, keepdims=True)
        picks.append(j)
        hit = cand == j
        Vw = jnp.where(hit, -jnp.inf, Vw)
        sel = jnp.logical_or(sel, hit)
    out_fast = jnp.concatenate(picks, axis=0)
    out_ref[...] = out_fast + b * _N

    srow = lax.broadcasted_iota(jnp.int32, sel.shape, 0)
    lastslot = lax.rem(srow, S) == S - 1
    susp = jnp.any(jnp.logical_and(sel, lastslot), axis=0,
                   keepdims=True)

    @pl.when(jnp.any(susp))
    def _():
        fiota = lax.broadcasted_iota(jnp.int32, negdT.shape, 0)
        ndw = negdT
        slow = []
        for _ in range(_K):
            m = jnp.max(ndw, axis=0, keepdims=True)
            cand = jnp.where(ndw >= m, fiota, _N)
            j = jnp.min(cand, axis=0, keepdims=True)
            slow.append(j)
            ndw = jnp.where(cand == j, -jnp.inf, ndw)
        out_slow = jnp.concatenate(slow, axis=0)
        out_ref[...] = jnp.where(susp, out_slow, out_fast) + b * _N


def _knn(ptsT4, pts):
    return pl.pallas_call(
        _knn_body,
        grid=(_B, _N // _M1),
        in_specs=[
            pl.BlockSpec((None, _N, 4), lambda b, i: (b, 0, 0)),
            pl.BlockSpec((None, _M1, 4), lambda b, i: (b, i, 0)),
            pl.BlockSpec((None, 3, _M1), lambda b, i: (b, 0, i)),
        ],
        out_specs=pl.BlockSpec((None, _K, _M1), lambda b, i: (b, 0, i)),
        out_shape=jax.ShapeDtypeStruct((_B, _K, _N), jnp.int32),
    )(ptsT4, ptsT4, pts)


def _table_body(f_ref, p_ref, Wk_ref, bk_ref, Wv_ref, bv_ref, Wr1_ref,
                out_ref):
    f = f_ref[...]
    WB = jnp.dot(Wr1_ref[...], Wk_ref[...], precision=_HP)
    cB = lax.dot_general(bk_ref[...], Wr1_ref[...],
                         (((1,), (1,)), ((), ())), precision=_HP)
    bv_row = lax.dot_general(f, WB, (((1,), (1,)), ((), ())),
                             precision=_HP) + cB
    v_row = lax.dot_general(f, Wv_ref[...], (((1,), (1,)), ((), ())),
                            precision=_HP) + bv_ref[...]
    pad = jnp.zeros((f.shape[0], _ROW_W - (_OFF_V + _C)), jnp.float32)
    out_ref[...] = jnp.concatenate([p_ref[...], bv_row, v_row, pad], axis=1)


def _table(featsT2, ptsT2, Wk, bk2, Wv, bv2, Wr1):
    BN = _B * _N
    full = lambda a: pl.BlockSpec(a.shape, lambda i: tuple(0 for _ in a.shape))
    return pl.pallas_call(
        _table_body,
        grid=(BN // _M2,),
        in_specs=[
            pl.BlockSpec((_M2, _C), lambda i: (i, 0)),
            pl.BlockSpec((_M2, 4), lambda i: (i, 0)),
            full(Wk), full(bk2), full(Wv), full(bv2), full(Wr1),
        ],
        out_specs=pl.BlockSpec((_M2, _ROW_W), lambda i: (i, 0)),
        out_shape=jax.ShapeDtypeStruct((BN, _ROW_W), jnp.float32),
    )(featsT2, ptsT2, Wk, bk2, Wv, bv2, Wr1)


_NC, _NS = 2, 16
_NW = _NC * _NS
_CH = 128
_ROWS_PER_W = (_B * _N * _K) // _NW
_NCHUNK = _ROWS_PER_W // _CH


def _gather_kernel(table_hbm, idx_hbm, out_hbm, idx_v, buf0, buf1,
                   sem0, sem1):
    wid = lax.axis_index("s") * _NC + lax.axis_index("c")
    base = wid * _ROWS_PER_W
    pltpu.sync_copy(idx_hbm.at[pl.ds(base, _ROWS_PER_W)], idx_v)
    bufs = (buf0, buf1)
    sems = (sem0, sem1)

    def issue(i, slot):
        return pltpu.async_copy(
            table_hbm.at[idx_v.at[pl.ds(i * _CH, _CH)]], bufs[slot],
            sems[slot])

    issue(0, 0)
    @pl.loop(0, _NCHUNK // 2)
    def _(g):
        i0 = g * 2

        @pl.when(i0 + 1 < _NCHUNK)
        def _():
            issue(i0 + 1, 1)
        pltpu.make_async_copy(table_hbm.at[idx_v.at[pl.ds(0, _CH)]], bufs[0],
                              sems[0]).wait()
        pltpu.sync_copy(bufs[0], out_hbm.at[pl.ds(base + i0 * _CH, _CH)])

        @pl.when(i0 + 2 < _NCHUNK)
        def _():
            issue(i0 + 2, 0)
        pltpu.make_async_copy(table_hbm.at[idx_v.at[pl.ds(0, _CH)]], bufs[1],
                              sems[1]).wait()
        pltpu.sync_copy(bufs[1], out_hbm.at[pl.ds(base + (i0 + 1) * _CH, _CH)])


def _gather(table, idx_flat):
    mesh = plsc.VectorSubcoreMesh(core_axis_name="c", subcore_axis_name="s")
    f = pl.kernel(
        _gather_kernel,
        out_type=jax.ShapeDtypeStruct((_B * _N * _K, _ROW_W), jnp.float32),
        mesh=mesh,
        scratch_types=[
            pltpu.VMEM((_ROWS_PER_W,), jnp.int32),
            pltpu.VMEM((_CH, _ROW_W), jnp.float32),
            pltpu.VMEM((_CH, _ROW_W), jnp.float32),
            pltpu.SemaphoreType.DMA,
            pltpu.SemaphoreType.DMA,
        ],
    )
    return f(table, idx_flat)


def _attn_body(g_ref, f_ref, p_ref, te_ref,
               Wq_ref, bq_ref, Wg1_ref, bg1_ref, Wg2_ref, bg2_ref,
               Wr1_ref, br1_ref, Wr2_ref, br2_ref,
               Wo1_ref, bo1_ref, Wo2_ref, bo2_ref,
               Wt_ref, bt_ref, Wtg_ref, btg_ref, Wtb_ref,
               out_ref):
    MK = _M3 * _K
    G = jnp.reshape(g_ref[...], (_K * _M3, _ROW_W))
    f = f_ref[...]
    Wr1 = Wr1_ref[...]

    pj = G[:, _OFF_P:_OFF_P + 4]
    pn = jnp.broadcast_to(p_ref[...][None, :, :], (_K, _M3, 4))
    gd = jnp.reshape(pn, (MK, 4)) - pj
    g1 = lax.dot_general(gd, Wg1_ref[...], (((1,), (1,)), ((), ())),
                         precision=_HP) + bg1_ref[...]
    g1 = jnp.maximum(g1, 0.0)
    ge = lax.dot_general(g1, Wg2_ref[...], (((1,), (1,)), ((), ())),
                         precision=_HP) + bg2_ref[...]

    WA = jnp.dot(Wr1, Wq_ref[...], precision=_HP)
    cA = lax.dot_general(bq_ref[...], Wr1, (((1,), (1,)), ((), ())),
                         precision=_HP)
    A = lax.dot_general(f, WA, (((1,), (1,)), ((), ())),
                        precision=_HP) + cA
    A = jnp.reshape(jnp.broadcast_to(A[None, :, :], (_K, _M3, 32)), (MK, 32))
    Wg2p = jnp.dot(Wr1, Wg2_ref[...], precision=_HP)
    cg = lax.dot_general(bg2_ref[...], Wr1, (((1,), (1,)), ((), ())),
                         precision=_HP)
    r1 = (A - G[:, _OFF_BV:_OFF_BV + 32]
          + lax.dot_general(g1, Wg2p, (((1,), (1,)), ((), ())),
                            precision=_HP)
          + cg + br1_ref[...])
    r1 = jnp.maximum(r1 * _BN_SCALE, 0.0)
    r2 = lax.dot_general(r1, Wr2_ref[...], (((1,), (1,)), ((), ())),
                         precision=_HP) + br2_ref[...]

    r3 = jnp.reshape(r2, (_K, _M3, _C))
    mx = jnp.max(r3, axis=0, keepdims=True)
    ex = jnp.exp(r3 - mx)
    attn = ex / jnp.sum(ex, axis=0, keepdims=True)

    vg = jnp.reshape(G[:, _OFF_V:_OFF_V + _C] + ge, (_K, _M3, _C))
    agg = jnp.sum(attn * vg, axis=0) + f

    o = lax.dot_general(agg, Wo1_ref[...], (((1,), (1,)), ((), ())),
                        precision=_HP) + bo1_ref[...]
    o = jnp.maximum(o, 0.0)
    o2 = lax.dot_general(o, Wo2_ref[...], (((1,), (1,)), ((), ())),
                         precision=_HP) + bo2_ref[...] + agg

    te = te_ref[...]
    gate = jax.nn.sigmoid(lax.dot_general(te, Wtg_ref[...],
                                          (((1,), (1,)), ((), ())),
                                          precision=_HP) + btg_ref[...])
    tbias = lax.dot_general(te, Wtb_ref[...], (((1,), (1,)), ((), ())),
                            precision=_HP)
    out = lax.dot_general(o2, Wt_ref[...], (((1,), (1,)), ((), ())),
                          precision=_HP) + bt_ref[...]
    out_ref[...] = out * gate + tbias


def _attn(gath4, featsT, ptsT4, te3, weights):
    full = lambda a: pl.BlockSpec(a.shape,
                                  lambda b, i: tuple(0 for _ in a.shape))
    w_specs = [full(w) for w in weights]
    return pl.pallas_call(
        _attn_body,
        grid=(_B, _N // _M3),
        in_specs=[
            pl.BlockSpec((None, _K, _M3, _ROW_W), lambda b, i: (b, 0, i, 0)),
            pl.BlockSpec((None, _M3, _C), lambda b, i: (b, i, 0)),
            pl.BlockSpec((None, _M3, 4), lambda b, i: (b, i, 0)),
            pl.BlockSpec((None, 1, _T), lambda b, i: (b, 0, 0)),
        ] + w_specs,
        out_specs=pl.BlockSpec((None, _M3, _C), lambda b, i: (b, i, 0)),
        out_shape=jax.ShapeDtypeStruct((_B, _N, _C), jnp.float32),
    )(gath4, featsT, ptsT4, te3, *weights)


def kernel(pts, feats, time_emb, Wq, bq, Wk, bk, Wv, bv, Wg1, bg1, Wg2, bg2,
           Wr1, br1, Wr2, br2, Wo1, bo1, Wo2, bo2, Wt, bt, Wtg, btg, Wtb):
    r2 = lambda x: jnp.reshape(x, (1, -1))
    ptsT4 = jnp.pad(jnp.transpose(pts, (0, 2, 1)), ((0, 0), (0, 0), (0, 1)))
    featsT = jnp.transpose(feats, (0, 2, 1))
    Wg1p = jnp.pad(Wg1, ((0, 0), (0, 1)))

    table = _table(jnp.reshape(featsT, (_B * _N, _C)),
                   jnp.reshape(ptsT4, (_B * _N, 4)),
                   Wk, r2(bk), Wv, r2(bv), Wr1)
    weights = [Wq, r2(bq), Wg1p, r2(bg1), Wg2, r2(bg2), Wr1, r2(br1),
               Wr2, r2(br2), Wo1, r2(bo1), Wo2, r2(bo2), Wt, r2(bt),
               Wtg, r2(btg), Wtb]
    te3 = jnp.reshape(time_emb, (_B, 1, _T))
    idx = _knn(ptsT4, pts)
    gath = _gather(table, jnp.reshape(idx, (-1,)))
    gath4 = jnp.reshape(gath, (_B, _K, _N, _ROW_W))
    outT = _attn(gath4, featsT, ptsT4, te3, weights)
    return jnp.transpose(outT, (0, 2, 1))


# speedup vs baseline: 1.1915x; 1.1915x over previous
import functools

import jax
import jax.numpy as jnp
import numpy as np
from jax import lax
from jax.experimental import pallas as pl
from jax.experimental.pallas import tpu as pltpu
from jax.experimental.pallas import tpu_sc as plsc

_B, _C, _N, _K, _T = 4, 64, 4096, 16, 128
_HP = lax.Precision.DEFAULT
_BN_SCALE = float(1.0 / np.sqrt(1.0 + 1e-5))

_ROW_W = 128
_OFF_P, _OFF_BV, _OFF_V = 0, 4, 36

_M1 = 128
_M2 = 1024
_M3 = 128


def _knn_body(colsT_ref, rowsT_ref, ptile_ref, out_ref):
    b = pl.program_id(0)
    colsT = colsT_ref[...]
    rows = rowsT_ref[...]
    ptile = ptile_ref[...]
    innerT = lax.dot_general(colsT, rows, (((1,), (1,)), ((), ())),
                             precision=lax.Precision.DEFAULT)
    sq_c = jnp.sum(colsT * colsT, axis=1, keepdims=True)
    sq_q = jnp.sum(ptile * ptile, axis=0, keepdims=True)
    d = (sq_c + sq_q) - 2.0 * innerT

    S, W, NB = 5, 128, _N // 128
    dk = lax.bitcast_convert_type(jnp.maximum(d, 0.0), jnp.int32)
    fiota = lax.broadcasted_iota(jnp.int32, d.shape, 0)
    keys = jnp.bitwise_or(jnp.bitwise_and(dk, jnp.int32(-128)),
                          jnp.bitwise_and(fiota, jnp.int32(127)))
    IMAX = jnp.int32(2**31 - 1)
    vparts, jparts = [], []
    for nb in range(NB):
        kb = keys[nb * W:(nb + 1) * W, :]
        sv, sj = [], []
        for _ in range(S):
            m = jnp.min(kb, axis=0, keepdims=True)
            sv.append(jnp.bitwise_and(m, jnp.int32(-128)))
            sj.append(jnp.bitwise_and(m, jnp.int32(127)) + nb * W)
            kb = jnp.where(kb == m, IMAX, kb)
        vparts.append(jnp.concatenate(sv, axis=0))
        jparts.append(jnp.concatenate(sj, axis=0))
    V = jnp.concatenate(vparts, axis=0)
    J = jnp.concatenate(jparts, axis=0)
    sel = jnp.zeros(V.shape, jnp.bool_)
    picks = []
    Vw = V
    BIG = jnp.int32(1 << 20)
    for _ in range(_K):
        m = jnp.min(Vw, axis=0, keepdims=True)
        cand = jnp.where(Vw == m, J, BIG)
        j = jnp.min(cand, axis=0, keepdims=True)
        picks.append(j)
        hit = cand == j
        Vw = jnp.where(hit, IMAX, Vw)
        sel = jnp.logical_or(sel, hit)
    out_fast = jnp.concatenate(picks, axis=0)
    out_ref[...] = out_fast + b * _N

    srow = lax.broadcasted_iota(jnp.int32, sel.shape, 0)
    lastslot = lax.rem(srow, S) == S - 1
    susp = jnp.any(jnp.logical_and(sel, lastslot), axis=0,
                   keepdims=True)

    @pl.when(jnp.any(susp))
    def _():
        ndw = d
        slow = []
        for _ in range(_K):
            m = jnp.min(ndw, axis=0, keepdims=True)
            cand = jnp.where(ndw <= m, fiota, _N)
            j = jnp.min(cand, axis=0, keepdims=True)
            slow.append(j)
            ndw = jnp.where(cand == j, jnp.float32(jnp.inf), ndw)
        out_slow = jnp.concatenate(slow, axis=0)
        out_ref[...] = jnp.where(susp, out_slow, out_fast) + b * _N


def _knn(ptsT4, pts):
    return pl.pallas_call(
        _knn_body,
        grid=(_B, _N // _M1),
        in_specs=[
            pl.BlockSpec((None, _N, 4), lambda b, i: (b, 0, 0)),
            pl.BlockSpec((None, _M1, 4), lambda b, i: (b, i, 0)),
            pl.BlockSpec((None, 3, _M1), lambda b, i: (b, 0, i)),
        ],
        out_specs=pl.BlockSpec((None, _K, _M1), lambda b, i: (b, 0, i)),
        out_shape=jax.ShapeDtypeStruct((_B, _K, _N), jnp.int32),
    )(ptsT4, ptsT4, pts)


def _table_body(f_ref, p_ref, Wk_ref, bk_ref, Wv_ref, bv_ref, Wr1_ref,
                out_ref):
    f = f_ref[...]
    WB = jnp.dot(Wr1_ref[...], Wk_ref[...], precision=_HP)
    cB = lax.dot_general(bk_ref[...], Wr1_ref[...],
                         (((1,), (1,)), ((), ())), precision=_HP)
    bv_row = lax.dot_general(f, WB, (((1,), (1,)), ((), ())),
                             precision=_HP) + cB
    v_row = lax.dot_general(f, Wv_ref[...], (((1,), (1,)), ((), ())),
                            precision=_HP) + bv_ref[...]
    pad = jnp.zeros((f.shape[0], _ROW_W - (_OFF_V + _C)), jnp.float32)
    out_ref[...] = jnp.concatenate([p_ref[...], bv_row, v_row, pad], axis=1)


def _table(featsT2, ptsT2, Wk, bk2, Wv, bv2, Wr1):
    BN = _B * _N
    full = lambda a: pl.BlockSpec(a.shape, lambda i: tuple(0 for _ in a.shape))
    return pl.pallas_call(
        _table_body,
        grid=(BN // _M2,),
        in_specs=[
            pl.BlockSpec((_M2, _C), lambda i: (i, 0)),
            pl.BlockSpec((_M2, 4), lambda i: (i, 0)),
            full(Wk), full(bk2), full(Wv), full(bv2), full(Wr1),
        ],
        out_specs=pl.BlockSpec((_M2, _ROW_W), lambda i: (i, 0)),
        out_shape=jax.ShapeDtypeStruct((BN, _ROW_W), jnp.float32),
    )(featsT2, ptsT2, Wk, bk2, Wv, bv2, Wr1)


_NC, _NS = 2, 16
_NW = _NC * _NS
_CH = 128
_ROWS_PER_W = (_B * _N * _K) // _NW
_NCHUNK = _ROWS_PER_W // _CH


def _gather_kernel(table_hbm, idx_hbm, out_hbm, idx_v, buf0, buf1,
                   sem0, sem1):
    wid = lax.axis_index("s") * _NC + lax.axis_index("c")
    base = wid * _ROWS_PER_W
    pltpu.sync_copy(idx_hbm.at[pl.ds(base, _ROWS_PER_W)], idx_v)
    bufs = (buf0, buf1)
    sems = (sem0, sem1)

    def issue(i, slot):
        return pltpu.async_copy(
            table_hbm.at[idx_v.at[pl.ds(i * _CH, _CH)]], bufs[slot],
            sems[slot])

    issue(0, 0)
    @pl.loop(0, _NCHUNK // 2)
    def _(g):
        i0 = g * 2

        @pl.when(i0 + 1 < _NCHUNK)
        def _():
            issue(i0 + 1, 1)
        pltpu.make_async_copy(table_hbm.at[idx_v.at[pl.ds(0, _CH)]], bufs[0],
                              sems[0]).wait()
        pltpu.sync_copy(bufs[0], out_hbm.at[pl.ds(base + i0 * _CH, _CH)])

        @pl.when(i0 + 2 < _NCHUNK)
        def _():
            issue(i0 + 2, 0)
        pltpu.make_async_copy(table_hbm.at[idx_v.at[pl.ds(0, _CH)]], bufs[1],
                              sems[1]).wait()
        pltpu.sync_copy(bufs[1], out_hbm.at[pl.ds(base + (i0 + 1) * _CH, _CH)])


def _gather(table, idx_flat):
    mesh = plsc.VectorSubcoreMesh(core_axis_name="c", subcore_axis_name="s")
    f = pl.kernel(
        _gather_kernel,
        out_type=jax.ShapeDtypeStruct((_B * _N * _K, _ROW_W), jnp.float32),
        mesh=mesh,
        scratch_types=[
            pltpu.VMEM((_ROWS_PER_W,), jnp.int32),
            pltpu.VMEM((_CH, _ROW_W), jnp.float32),
            pltpu.VMEM((_CH, _ROW_W), jnp.float32),
            pltpu.SemaphoreType.DMA,
            pltpu.SemaphoreType.DMA,
        ],
    )
    return f(table, idx_flat)


def _attn_body(g_ref, f_ref, p_ref, te_ref,
               Wq_ref, bq_ref, Wg1_ref, bg1_ref, Wg2_ref, bg2_ref,
               Wr1_ref, br1_ref, Wr2_ref, br2_ref,
               Wo1_ref, bo1_ref, Wo2_ref, bo2_ref,
               Wt_ref, bt_ref, Wtg_ref, btg_ref, Wtb_ref,
               out_ref):
    MK = _M3 * _K
    G = jnp.reshape(g_ref[...], (_K * _M3, _ROW_W))
    f = f_ref[...]
    Wr1 = Wr1_ref[...]

    pj = G[:, _OFF_P:_OFF_P + 4]
    pn = jnp.broadcast_to(p_ref[...][None, :, :], (_K, _M3, 4))
    gd = jnp.reshape(pn, (MK, 4)) - pj
    g1 = lax.dot_general(gd, Wg1_ref[...], (((1,), (1,)), ((), ())),
                         precision=_HP) + bg1_ref[...]
    g1 = jnp.maximum(g1, 0.0)
    ge = lax.dot_general(g1, Wg2_ref[...], (((1,), (1,)), ((), ())),
                         precision=_HP) + bg2_ref[...]

    WA = jnp.dot(Wr1, Wq_ref[...], precision=_HP)
    cA = lax.dot_general(bq_ref[...], Wr1, (((1,), (1,)), ((), ())),
                         precision=_HP)
    A = lax.dot_general(f, WA, (((1,), (1,)), ((), ())),
                        precision=_HP) + cA
    A = jnp.reshape(jnp.broadcast_to(A[None, :, :], (_K, _M3, 32)), (MK, 32))
    Wg2p = jnp.dot(Wr1, Wg2_ref[...], precision=_HP)
    cg = lax.dot_general(bg2_ref[...], Wr1, (((1,), (1,)), ((), ())),
                         precision=_HP)
    r1 = (A - G[:, _OFF_BV:_OFF_BV + 32]
          + lax.dot_general(g1, Wg2p, (((1,), (1,)), ((), ())),
                            precision=_HP)
          + cg + br1_ref[...])
    r1 = jnp.maximum(r1 * _BN_SCALE, 0.0)
    r2 = lax.dot_general(r1, Wr2_ref[...], (((1,), (1,)), ((), ())),
                         precision=_HP) + br2_ref[...]

    r3 = jnp.reshape(r2, (_K, _M3, _C))
    mx = jnp.max(r3, axis=0, keepdims=True)
    ex = jnp.exp(r3 - mx)
    attn = ex / jnp.sum(ex, axis=0, keepdims=True)

    vg = jnp.reshape(G[:, _OFF_V:_OFF_V + _C] + ge, (_K, _M3, _C))
    agg = jnp.sum(attn * vg, axis=0) + f

    o = lax.dot_general(agg, Wo1_ref[...], (((1,), (1,)), ((), ())),
                        precision=_HP) + bo1_ref[...]
    o = jnp.maximum(o, 0.0)
    o2 = lax.dot_general(o, Wo2_ref[...], (((1,), (1,)), ((), ())),
                         precision=_HP) + bo2_ref[...] + agg

    te = te_ref[...]
    gate = jax.nn.sigmoid(lax.dot_general(te, Wtg_ref[...],
                                          (((1,), (1,)), ((), ())),
                                          precision=_HP) + btg_ref[...])
    tbias = lax.dot_general(te, Wtb_ref[...], (((1,), (1,)), ((), ())),
                            precision=_HP)
    out = lax.dot_general(o2, Wt_ref[...], (((1,), (1,)), ((), ())),
                          precision=_HP) + bt_ref[...]
    out_ref[...] = out * gate + tbias


def _attn(gath4, featsT, ptsT4, te3, weights):
    full = lambda a: pl.BlockSpec(a.shape,
                                  lambda b, i: tuple(0 for _ in a.shape))
    w_specs = [full(w) for w in weights]
    return pl.pallas_call(
        _attn_body,
        grid=(_B, _N // _M3),
        in_specs=[
            pl.BlockSpec((None, _K, _M3, _ROW_W), lambda b, i: (b, 0, i, 0)),
            pl.BlockSpec((None, _M3, _C), lambda b, i: (b, i, 0)),
            pl.BlockSpec((None, _M3, 4), lambda b, i: (b, i, 0)),
            pl.BlockSpec((None, 1, _T), lambda b, i: (b, 0, 0)),
        ] + w_specs,
        out_specs=pl.BlockSpec((None, _M3, _C), lambda b, i: (b, i, 0)),
        out_shape=jax.ShapeDtypeStruct((_B, _N, _C), jnp.float32),
    )(gath4, featsT, ptsT4, te3, *weights)


def kernel(pts, feats, time_emb, Wq, bq, Wk, bk, Wv, bv, Wg1, bg1, Wg2, bg2,
           Wr1, br1, Wr2, br2, Wo1, bo1, Wo2, bo2, Wt, bt, Wtg, btg, Wtb):
    r2 = lambda x: jnp.reshape(x, (1, -1))
    ptsT4 = jnp.pad(jnp.transpose(pts, (0, 2, 1)), ((0, 0), (0, 0), (0, 1)))
    featsT = jnp.transpose(feats, (0, 2, 1))
    Wg1p = jnp.pad(Wg1, ((0, 0), (0, 1)))

    table = _table(jnp.reshape(featsT, (_B * _N, _C)),
                   jnp.reshape(ptsT4, (_B * _N, 4)),
                   Wk, r2(bk), Wv, r2(bv), Wr1)
    weights = [Wq, r2(bq), Wg1p, r2(bg1), Wg2, r2(bg2), Wr1, r2(br1),
               Wr2, r2(br2), Wo1, r2(bo1), Wo2, r2(bo2), Wt, r2(bt),
               Wtg, r2(btg), Wtb]
    te3 = jnp.reshape(time_emb, (_B, 1, _T))
    idx = _knn(ptsT4, pts)
    gath = _gather(table, jnp.reshape(idx, (-1,)))
    gath4 = jnp.reshape(gath, (_B, _K, _N, _ROW_W))
    outT = _attn(gath4, featsT, ptsT4, te3, weights)
    return jnp.transpose(outT, (0, 2, 1))
